# baseline (speedup 1.0000x reference)
import dataclasses
import functools

import jax
import jax.numpy as jnp
import numpy as np
from jax import lax
from jax.experimental import pallas as pl
from jax.experimental.pallas import tpu as pltpu
from jax.experimental.pallas import tpu_sc as plsc

GRID = 26
S = GRID * GRID
A = 5
B = 1024
SA = S * A
LANE = SA * A
NROW128 = B * LANE // 128
DC = 3200
DR = B * LANE // DC
BR = 416
NSTEPS = DR // BR

_ch = np.arange(DC) % 5
_C1 = jnp.asarray(np.where(_ch < 2, 1.25 / 2704.0, 0.0)[None, :], jnp.float32)
_C3 = jnp.asarray(np.where(_ch == 4, 0.5, 0.0)[None, :], jnp.float32)


def _dense_body(pred_ref, c1_ref, c2_ref, c3_ref, obj_ref, out_ref):
    i = pl.program_id(0)
    v = pred_ref[...]
    e = jnp.exp(v)
    r = 1.0 / (e + 1.0)
    u = e - 1.0
    w = 1.0 - r
    p = c1_ref[...] * (r * r) + c2_ref[...]
    elem = (u * u) * p + c3_ref[...] * (w * w)
    s = jnp.sum(elem)
    acc = jnp.where(i == 0, 0.0, out_ref[0, 0]) + s

    @pl.when(i < NSTEPS - 1)
    def _():
        out_ref[0, 0] = acc

    @pl.when(i == NSTEPS - 1)
    def _():
        ot = obj_ref[...]
        co_sum = jnp.sum(ot[0, :])
        cf_sum = jnp.sum(ot[1, :])
        corr_sum = jnp.sum(ot[2, :])
        out_ref[0, 0] = (5.0 / (4.0 * B)) * co_sum + cf_sum / B \
            + (acc - corr_sum) / (B * (SA - 1))


def _sc_objterms(pred16, lab_t, awp, ahp):
    mesh = plsc.VectorSubcoreMesh(core_axis_name="c", subcore_axis_name="s")
    cp = pltpu.CompilerParams()
    if "needs_layout_passes" in pltpu.CompilerParams.__dataclass_fields__:
        cp = dataclasses.replace(cp, needs_layout_passes=False)

    @functools.partial(
        pl.kernel,
        mesh=mesh,
        compiler_params=cp,
        out_type=jax.ShapeDtypeStruct((3, B), jnp.float32),
        scratch_types=[
            pltpu.VMEM((4, 32), jnp.float32),
            pltpu.VMEM((16,), jnp.float32),
            pltpu.VMEM((16,), jnp.float32),
            pltpu.VMEM((16,), jnp.int32),
            pltpu.VMEM((16,), jnp.int32),
            pltpu.VMEM((16, 128), jnp.float32),
            pltpu.VMEM((16, 128), jnp.float32),
            pltpu.VMEM((3, 32), jnp.float32),
            pltpu.SemaphoreType.DMA,
        ],
    )
    def body(pred_hbm, lab_hbm, aw_hbm, ah_hbm, out_hbm,
             lab_v, aw_v, ah_v, idx_a, idx_b, buf_a, buf_b, out_v, sem):
        wid = lax.axis_index("s") * 2 + lax.axis_index("c")
        base = wid * 32
        for c in range(4):
            pltpu.sync_copy(lab_hbm.at[c, pl.ds(base, 32)], lab_v.at[c])
        pltpu.sync_copy(aw_hbm, aw_v)
        pltpu.sync_copy(ah_hbm, ah_v)
        iota = lax.iota(jnp.int32, 16)

        for g in range(2):
            sl = pl.ds(g * 16, 16)
            lx = lab_v[0, sl]
            ly = lab_v[1, sl]
            lw = lab_v[2, sl]
            lh = lab_v[3, sl]
            ixf = (lx * float(GRID)).astype(jnp.int32)
            iyf = (ly * float(GRID)).astype(jnp.int32)
            s_obj = ixf * GRID + iyf
            awvec = aw_v[...]
            ahvec = ah_v[...]
            best_a = jnp.zeros((16,), jnp.int32)
            best_d = None
            for a in range(A):
                awa = jnp.sum(jnp.where(iota == a, awvec, 0.0))
                aha = jnp.sum(jnp.where(iota == a, ahvec, 0.0))
                d = (lw - awa) * (lw - awa) + (lh - aha) * (lh - aha)
                if best_d is None:
                    best_d = d
                else:
                    m = d > best_d
                    best_a = jnp.where(m, a, best_a)
                    best_d = jnp.where(m, d, best_d)
            slot = s_obj * A + best_a
            b_vec = base + g * 16 + iota
            eb = b_vec * LANE + slot * A
            r0 = eb >> 7
            r1 = jnp.minimum(r0 + 1, NROW128 - 1)
            off = eb - (r0 << 7)
            idx_a[...] = r0
            idx_b[...] = r1
            pltpu.async_copy(pred_hbm.at[idx_a], buf_a, sem).wait()
            pltpu.async_copy(pred_hbm.at[idx_b], buf_b, sem).wait()
            pv = []
            for c in range(5):
                oc = off + c
                va = plsc.load_gather(buf_a, [iota, jnp.minimum(oc, 127)])
                vb = plsc.load_gather(buf_b, [iota, jnp.maximum(oc - 128, 0)])
                pv.append(jnp.where(oc < 128, va, vb))
            p0, p1, p2, p3, p4 = pv
            e0 = jnp.exp(p0)
            e1 = jnp.exp(p1)
            e2 = jnp.exp(p2)
            e3 = jnp.exp(p3)
            e4 = jnp.exp(p4)
            r0f = 1.0 / (e0 + 1.0)
            r1f = 1.0 / (e1 + 1.0)
            sig4 = 1.0 - 1.0 / (e4 + 1.0)
            aw_o = plsc.load_gather(aw_v, [best_a])
            ah_o = plsc.load_gather(ah_v, [best_a])
            px = (ixf.astype(jnp.float32) + (1.0 - r0f)) / float(GRID)
            py = (iyf.astype(jnp.float32) + (1.0 - r1f)) / float(GRID)
            pw = aw_o * e2
            ph = ah_o * e3
            co = ((px - lx) * (px - lx) + (py - ly) * (py - ly)
                  + (pw - lw) * (pw - lw) + (ph - lh) * (ph - lh))
            lx0 = jnp.maximum(lx - lw * 0.5, 0.0)
            ly0 = jnp.maximum(ly - lh * 0.5, 0.0)
            lx1 = jnp.minimum(lx + lw * 0.5, 1.0)
            ly1 = jnp.minimum(ly + lh * 0.5, 1.0)
            px0 = jnp.maximum(px - pw * 0.5, 0.0)
            py0 = jnp.maximum(py - ph * 0.5, 0.0)
            px1 = jnp.minimum(px + pw * 0.5, 1.0)
            py1 = jnp.minimum(py + ph * 0.5, 1.0)
            inter = (jnp.maximum(jnp.minimum(lx1, px1) - jnp.maximum(lx0, px0), 0.0)
                     * jnp.maximum(jnp.minimum(ly1, py1) - jnp.maximum(ly0, py0), 0.0))
            iou = inter / (lw * lh + pw * ph - inter)
            cf = (sig4 - iou) * (sig4 - iou)
            u0 = e0 - 1.0
            u1 = e1 - 1.0
            t0 = u0 * r0f
            t1 = u1 * r1f
            uw = aw_o * (e2 - 1.0)
            uh = ah_o * (e3 - 1.0)
            corr = (1.25 * ((t0 * t0 + t1 * t1) * (1.0 / 2704.0)
                            + uw * uw + uh * uh)
                    + 0.5 * (sig4 * sig4))
            out_v[0, sl] = co
            out_v[1, sl] = cf
            out_v[2, sl] = corr
        for k in range(3):
            pltpu.sync_copy(out_v.at[k], out_hbm.at[k, pl.ds(base, 32)])

    return body(pred16, lab_t, awp, ahp)


def kernel(pred_, label, anchors):
    pred2d = pred_.reshape(DR, DC)
    pred16 = pred_.reshape(NROW128, 128)
    lab_t = label.T
    aw = anchors[:, 0]
    ah = anchors[:, 1]
    awp = jnp.concatenate([aw, jnp.zeros((11,), jnp.float32)])
    ahp = jnp.concatenate([ah, jnp.zeros((11,), jnp.float32)])
    pat = jnp.zeros((25,), jnp.float32)
    pat = pat.at[jnp.arange(5) * 5 + 2].set(1.25 * aw * aw)
    pat = pat.at[jnp.arange(5) * 5 + 3].set(1.25 * ah * ah)
    c2 = jnp.tile(pat, DC // 25)[None, :]

    obj = _sc_objterms(pred16, lab_t, awp, ahp)

    out = pl.pallas_call(
        _dense_body,
        grid=(NSTEPS,),
        in_specs=[
            pl.BlockSpec((BR, DC), lambda i: (i, 0)),
            pl.BlockSpec((1, DC), lambda i: (0, 0)),
            pl.BlockSpec((1, DC), lambda i: (0, 0)),
            pl.BlockSpec((1, DC), lambda i: (0, 0)),
            pl.BlockSpec((3, B), lambda i: (0, 0)),
        ],
        out_specs=pl.BlockSpec(
            (1, 1), lambda i: (0, 0), memory_space=pltpu.SMEM),
        out_shape=jax.ShapeDtypeStruct((1, 1), jnp.float32),
        compiler_params=pltpu.CompilerParams(
            dimension_semantics=("arbitrary",)),
    )(pred2d, _C1, c2, _C3, obj)
    return out.reshape(())


# speedup vs baseline: 1.6446x; 1.6446x over previous
import dataclasses
import functools

import jax
import jax.numpy as jnp
import numpy as np
from jax import lax
from jax.experimental import pallas as pl
from jax.experimental.pallas import tpu as pltpu
from jax.experimental.pallas import tpu_sc as plsc

GRID = 26
S = GRID * GRID
A = 5
B = 1024
SA = S * A
LANE = SA * A
NROW128 = B * LANE // 128
BR = 10400
NSTEPS = NROW128 // BR

_T = (128 * np.arange(25)[:, None] + np.arange(128)[None, :]) % 25
_CH = _T % 5
_ANC = _T // 5
_C1_NP = np.where(_CH < 2, 1.25 / 2704.0, 0.0).astype(np.float32)
_C3_NP = np.where(_CH == 4, 0.5, 0.0).astype(np.float32)


def _dense_body(pred_ref, c1_ref, c2_ref, c3_ref, obj_ref, out_ref):
    i = pl.program_id(0)
    v = pred_ref[...]
    e = jnp.exp(v)
    r = 1.0 / (e + 1.0)
    u = e - 1.0
    w = 1.0 - r
    p = c1_ref[...] * (r * r) + c2_ref[...]
    elem = (u * u) * p + c3_ref[...] * (w * w)
    s = jnp.sum(elem)
    acc = jnp.where(i == 0, 0.0, out_ref[0, 0]) + s

    @pl.when(i < NSTEPS - 1)
    def _():
        out_ref[0, 0] = acc

    @pl.when(i == NSTEPS - 1)
    def _():
        ot = obj_ref[...]
        co_sum = jnp.sum(ot[0, :])
        cf_sum = jnp.sum(ot[1, :])
        corr_sum = jnp.sum(ot[2, :])
        out_ref[0, 0] = (5.0 / (4.0 * B)) * co_sum + cf_sum / B \
            + (acc - corr_sum) / (B * (SA - 1))


def _sc_objterms(pred16, lab_t, awp, ahp):
    mesh = plsc.VectorSubcoreMesh(core_axis_name="c", subcore_axis_name="s")
    cp = pltpu.CompilerParams()
    if "needs_layout_passes" in pltpu.CompilerParams.__dataclass_fields__:
        cp = dataclasses.replace(cp, needs_layout_passes=False)

    @functools.partial(
        pl.kernel,
        mesh=mesh,
        compiler_params=cp,
        out_type=jax.ShapeDtypeStruct((3, B), jnp.float32),
        scratch_types=[
            pltpu.VMEM((4, 32), jnp.float32),
            pltpu.VMEM((16,), jnp.float32),
            pltpu.VMEM((16,), jnp.float32),
            pltpu.VMEM((16,), jnp.int32),
            pltpu.VMEM((16,), jnp.int32),
            pltpu.VMEM((16, 128), jnp.float32),
            pltpu.VMEM((16, 128), jnp.float32),
            pltpu.VMEM((3, 32), jnp.float32),
            pltpu.SemaphoreType.DMA,
        ],
    )
    def body(pred_hbm, lab_hbm, aw_hbm, ah_hbm, out_hbm,
             lab_v, aw_v, ah_v, idx_a, idx_b, buf_a, buf_b, out_v, sem):
        wid = lax.axis_index("s") * 2 + lax.axis_index("c")
        base = wid * 32
        for c in range(4):
            pltpu.sync_copy(lab_hbm.at[c, pl.ds(base, 32)], lab_v.at[c])
        pltpu.sync_copy(aw_hbm, aw_v)
        pltpu.sync_copy(ah_hbm, ah_v)
        iota = lax.iota(jnp.int32, 16)

        for g in range(2):
            sl = pl.ds(g * 16, 16)
            lx = lab_v[0, sl]
            ly = lab_v[1, sl]
            lw = lab_v[2, sl]
            lh = lab_v[3, sl]
            ixf = (lx * float(GRID)).astype(jnp.int32)
            iyf = (ly * float(GRID)).astype(jnp.int32)
            s_obj = ixf * GRID + iyf
            awvec = aw_v[...]
            ahvec = ah_v[...]
            best_a = jnp.zeros((16,), jnp.int32)
            best_d = None
            for a in range(A):
                awa = jnp.sum(jnp.where(iota == a, awvec, 0.0))
                aha = jnp.sum(jnp.where(iota == a, ahvec, 0.0))
                d = (lw - awa) * (lw - awa) + (lh - aha) * (lh - aha)
                if best_d is None:
                    best_d = d
                else:
                    m = d > best_d
                    best_a = jnp.where(m, a, best_a)
                    best_d = jnp.where(m, d, best_d)
            slot = s_obj * A + best_a
            b_vec = base + g * 16 + iota
            eb = b_vec * LANE + slot * A
            r0 = eb >> 7
            r1 = jnp.minimum(r0 + 1, NROW128 - 1)
            off = eb - (r0 << 7)
            idx_a[...] = r0
            idx_b[...] = r1
            pltpu.async_copy(pred_hbm.at[idx_a], buf_a, sem).wait()
            pltpu.async_copy(pred_hbm.at[idx_b], buf_b, sem).wait()
            pv = []
            for c in range(5):
                oc = off + c
                va = plsc.load_gather(buf_a, [iota, jnp.minimum(oc, 127)])
                vb = plsc.load_gather(buf_b, [iota, jnp.maximum(oc - 128, 0)])
                pv.append(jnp.where(oc < 128, va, vb))
            p0, p1, p2, p3, p4 = pv
            e0 = jnp.exp(p0)
            e1 = jnp.exp(p1)
            e2 = jnp.exp(p2)
            e3 = jnp.exp(p3)
            e4 = jnp.exp(p4)
            r0f = 1.0 / (e0 + 1.0)
            r1f = 1.0 / (e1 + 1.0)
            sig4 = 1.0 - 1.0 / (e4 + 1.0)
            aw_o = plsc.load_gather(aw_v, [best_a])
            ah_o = plsc.load_gather(ah_v, [best_a])
            px = (ixf.astype(jnp.float32) + (1.0 - r0f)) / float(GRID)
            py = (iyf.astype(jnp.float32) + (1.0 - r1f)) / float(GRID)
            pw = aw_o * e2
            ph = ah_o * e3
            co = ((px - lx) * (px - lx) + (py - ly) * (py - ly)
                  + (pw - lw) * (pw - lw) + (ph - lh) * (ph - lh))
            lx0 = jnp.maximum(lx - lw * 0.5, 0.0)
            ly0 = jnp.maximum(ly - lh * 0.5, 0.0)
            lx1 = jnp.minimum(lx + lw * 0.5, 1.0)
            ly1 = jnp.minimum(ly + lh * 0.5, 1.0)
            px0 = jnp.maximum(px - pw * 0.5, 0.0)
            py0 = jnp.maximum(py - ph * 0.5, 0.0)
            px1 = jnp.minimum(px + pw * 0.5, 1.0)
            py1 = jnp.minimum(py + ph * 0.5, 1.0)
            inter = (jnp.maximum(jnp.minimum(lx1, px1) - jnp.maximum(lx0, px0), 0.0)
                     * jnp.maximum(jnp.minimum(ly1, py1) - jnp.maximum(ly0, py0), 0.0))
            iou = inter / (lw * lh + pw * ph - inter)
            cf = (sig4 - iou) * (sig4 - iou)
            u0 = e0 - 1.0
            u1 = e1 - 1.0
            t0 = u0 * r0f
            t1 = u1 * r1f
            uw = aw_o * (e2 - 1.0)
            uh = ah_o * (e3 - 1.0)
            corr = (1.25 * ((t0 * t0 + t1 * t1) * (1.0 / 2704.0)
                            + uw * uw + uh * uh)
                    + 0.5 * (sig4 * sig4))
            out_v[0, sl] = co
            out_v[1, sl] = cf
            out_v[2, sl] = corr
        for k in range(3):
            pltpu.sync_copy(out_v.at[k], out_hbm.at[k, pl.ds(base, 32)])

    return body(pred16, lab_t, awp, ahp)


def kernel(pred_, label, anchors):
    pred16 = pred_.reshape(NROW128, 128)
    lab_t = label.T
    aw = anchors[:, 0]
    ah = anchors[:, 1]
    awp = jnp.concatenate([aw, jnp.zeros((11,), jnp.float32)])
    ahp = jnp.concatenate([ah, jnp.zeros((11,), jnp.float32)])
    aw2 = 1.25 * aw * aw
    ah2 = 1.25 * ah * ah
    anc = jnp.asarray(_ANC)
    c2t = jnp.where(jnp.asarray(_CH == 2), aw2[anc], 0.0) \
        + jnp.where(jnp.asarray(_CH == 3), ah2[anc], 0.0)
    c1 = jnp.tile(jnp.asarray(_C1_NP), (BR // 25, 1))
    c2 = jnp.tile(c2t.astype(jnp.float32), (BR // 25, 1))
    c3 = jnp.tile(jnp.asarray(_C3_NP), (BR // 25, 1))

    obj = _sc_objterms(pred16, lab_t, awp, ahp)

    out = pl.pallas_call(
        _dense_body,
        grid=(NSTEPS,),
        in_specs=[
            pl.BlockSpec((BR, 128), lambda i: (i, 0)),
            pl.BlockSpec((BR, 128), lambda i: (0, 0)),
            pl.BlockSpec((BR, 128), lambda i: (0, 0)),
            pl.BlockSpec((BR, 128), lambda i: (0, 0)),
            pl.BlockSpec((3, B), lambda i: (0, 0)),
        ],
        out_specs=pl.BlockSpec(
            (1, 1), lambda i: (0, 0), memory_space=pltpu.SMEM),
        out_shape=jax.ShapeDtypeStruct((1, 1), jnp.float32),
        compiler_params=pltpu.CompilerParams(
            dimension_semantics=("arbitrary",)),
    )(pred16, c1, c2, c3, obj)
    return out.reshape(())


# speedup vs baseline: 1.6783x; 1.0204x over previous
import dataclasses
import functools

import jax
import jax.numpy as jnp
import numpy as np
from jax import lax
from jax.experimental import pallas as pl
from jax.experimental.pallas import tpu as pltpu
from jax.experimental.pallas import tpu_sc as plsc

GRID = 26
S = GRID * GRID
A = 5
B = 1024
SA = S * A
LANE = SA * A
NROW128 = B * LANE // 128
BR = 10400
NSTEPS = NROW128 // BR

_T = (128 * np.arange(25)[:, None] + np.arange(128)[None, :]) % 25
_CH = _T % 5
_ANC = _T // 5
_C1_NP = np.where(_CH < 2, 1.25 / 2704.0, 0.0).astype(np.float32)
_C3_NP = np.where(_CH == 4, 0.5, 0.0).astype(np.float32)


def _dense_body(pred_ref, c1_ref, c2_ref, c3_ref, obj_ref, out_ref):
    i = pl.program_id(0)
    v = pred_ref[...].reshape(BR // 200, 200, 128)
    e = jnp.exp(v)
    r = 1.0 / (e + 1.0)
    u = e - 1.0
    w = 1.0 - r
    p = c1_ref[...][None] * (r * r) + c2_ref[...][None]
    elem = (u * u) * p + c3_ref[...][None] * (w * w)
    s = jnp.sum(elem)
    acc = jnp.where(i == 0, 0.0, out_ref[0, 0]) + s

    @pl.when(i < NSTEPS - 1)
    def _():
        out_ref[0, 0] = acc

    @pl.when(i == NSTEPS - 1)
    def _():
        ot = obj_ref[...]
        co_sum = jnp.sum(ot[0, :])
        cf_sum = jnp.sum(ot[1, :])
        corr_sum = jnp.sum(ot[2, :])
        out_ref[0, 0] = (5.0 / (4.0 * B)) * co_sum + cf_sum / B \
            + (acc - corr_sum) / (B * (SA - 1))


def _sc_objterms(pred16, lab_t, awp, ahp):
    mesh = plsc.VectorSubcoreMesh(core_axis_name="c", subcore_axis_name="s")
    cp = pltpu.CompilerParams()
    if "needs_layout_passes" in pltpu.CompilerParams.__dataclass_fields__:
        cp = dataclasses.replace(cp, needs_layout_passes=False)

    @functools.partial(
        pl.kernel,
        mesh=mesh,
        compiler_params=cp,
        out_type=jax.ShapeDtypeStruct((3, B), jnp.float32),
        scratch_types=[
            pltpu.VMEM((4, 32), jnp.float32),
            pltpu.VMEM((16,), jnp.float32),
            pltpu.VMEM((16,), jnp.float32),
            pltpu.VMEM((16,), jnp.int32),
            pltpu.VMEM((16,), jnp.int32),
            pltpu.VMEM((16, 128), jnp.float32),
            pltpu.VMEM((16, 128), jnp.float32),
            pltpu.VMEM((3, 32), jnp.float32),
            pltpu.SemaphoreType.DMA,
        ],
    )
    def body(pred_hbm, lab_hbm, aw_hbm, ah_hbm, out_hbm,
             lab_v, aw_v, ah_v, idx_a, idx_b, buf_a, buf_b, out_v, sem):
        wid = lax.axis_index("s") * 2 + lax.axis_index("c")
        base = wid * 32
        for c in range(4):
            pltpu.sync_copy(lab_hbm.at[c, pl.ds(base, 32)], lab_v.at[c])
        pltpu.sync_copy(aw_hbm, aw_v)
        pltpu.sync_copy(ah_hbm, ah_v)
        iota = lax.iota(jnp.int32, 16)

        for g in range(2):
            sl = pl.ds(g * 16, 16)
            lx = lab_v[0, sl]
            ly = lab_v[1, sl]
            lw = lab_v[2, sl]
            lh = lab_v[3, sl]
            ixf = (lx * float(GRID)).astype(jnp.int32)
            iyf = (ly * float(GRID)).astype(jnp.int32)
            s_obj = ixf * GRID + iyf
            awvec = aw_v[...]
            ahvec = ah_v[...]
            best_a = jnp.zeros((16,), jnp.int32)
            best_d = None
            for a in range(A):
                awa = jnp.sum(jnp.where(iota == a, awvec, 0.0))
                aha = jnp.sum(jnp.where(iota == a, ahvec, 0.0))
                d = (lw - awa) * (lw - awa) + (lh - aha) * (lh - aha)
                if best_d is None:
                    best_d = d
                else:
                    m = d > best_d
                    best_a = jnp.where(m, a, best_a)
                    best_d = jnp.where(m, d, best_d)
            slot = s_obj * A + best_a
            b_vec = base + g * 16 + iota
            eb = b_vec * LANE + slot * A
            r0 = eb >> 7
            r1 = jnp.minimum(r0 + 1, NROW128 - 1)
            off = eb - (r0 << 7)
            idx_a[...] = r0
            idx_b[...] = r1
            pltpu.async_copy(pred_hbm.at[idx_a], buf_a, sem).wait()
            pltpu.async_copy(pred_hbm.at[idx_b], buf_b, sem).wait()
            pv = []
            for c in range(5):
                oc = off + c
                va = plsc.load_gather(buf_a, [iota, jnp.minimum(oc, 127)])
                vb = plsc.load_gather(buf_b, [iota, jnp.maximum(oc - 128, 0)])
                pv.append(jnp.where(oc < 128, va, vb))
            p0, p1, p2, p3, p4 = pv
            e0 = jnp.exp(p0)
            e1 = jnp.exp(p1)
            e2 = jnp.exp(p2)
            e3 = jnp.exp(p3)
            e4 = jnp.exp(p4)
            r0f = 1.0 / (e0 + 1.0)
            r1f = 1.0 / (e1 + 1.0)
            sig4 = 1.0 - 1.0 / (e4 + 1.0)
            aw_o = plsc.load_gather(aw_v, [best_a])
            ah_o = plsc.load_gather(ah_v, [best_a])
            px = (ixf.astype(jnp.float32) + (1.0 - r0f)) / float(GRID)
            py = (iyf.astype(jnp.float32) + (1.0 - r1f)) / float(GRID)
            pw = aw_o * e2
            ph = ah_o * e3
            co = ((px - lx) * (px - lx) + (py - ly) * (py - ly)
                  + (pw - lw) * (pw - lw) + (ph - lh) * (ph - lh))
            lx0 = jnp.maximum(lx - lw * 0.5, 0.0)
            ly0 = jnp.maximum(ly - lh * 0.5, 0.0)
            lx1 = jnp.minimum(lx + lw * 0.5, 1.0)
            ly1 = jnp.minimum(ly + lh * 0.5, 1.0)
            px0 = jnp.maximum(px - pw * 0.5, 0.0)
            py0 = jnp.maximum(py - ph * 0.5, 0.0)
            px1 = jnp.minimum(px + pw * 0.5, 1.0)
            py1 = jnp.minimum(py + ph * 0.5, 1.0)
            inter = (jnp.maximum(jnp.minimum(lx1, px1) - jnp.maximum(lx0, px0), 0.0)
                     * jnp.maximum(jnp.minimum(ly1, py1) - jnp.maximum(ly0, py0), 0.0))
            iou = inter / (lw * lh + pw * ph - inter)
            cf = (sig4 - iou) * (sig4 - iou)
            u0 = e0 - 1.0
            u1 = e1 - 1.0
            t0 = u0 * r0f
            t1 = u1 * r1f
            uw = aw_o * (e2 - 1.0)
            uh = ah_o * (e3 - 1.0)
            corr = (1.25 * ((t0 * t0 + t1 * t1) * (1.0 / 2704.0)
                            + uw * uw + uh * uh)
                    + 0.5 * (sig4 * sig4))
            out_v[0, sl] = co
            out_v[1, sl] = cf
            out_v[2, sl] = corr
        for k in range(3):
            pltpu.sync_copy(out_v.at[k], out_hbm.at[k, pl.ds(base, 32)])

    return body(pred16, lab_t, awp, ahp)


def kernel(pred_, label, anchors):
    pred16 = pred_.reshape(NROW128, 128)
    lab_t = label.T
    aw = anchors[:, 0]
    ah = anchors[:, 1]
    awp = jnp.concatenate([aw, jnp.zeros((11,), jnp.float32)])
    ahp = jnp.concatenate([ah, jnp.zeros((11,), jnp.float32)])
    aw2 = 1.25 * aw * aw
    ah2 = 1.25 * ah * ah
    anc = jnp.asarray(_ANC)
    c2t = jnp.where(jnp.asarray(_CH == 2), aw2[anc], 0.0) \
        + jnp.where(jnp.asarray(_CH == 3), ah2[anc], 0.0)
    c1 = jnp.tile(jnp.asarray(_C1_NP), (8, 1))
    c2 = jnp.tile(c2t.astype(jnp.float32), (8, 1))
    c3 = jnp.tile(jnp.asarray(_C3_NP), (8, 1))

    obj = _sc_objterms(pred16, lab_t, awp, ahp)

    out = pl.pallas_call(
        _dense_body,
        grid=(NSTEPS,),
        in_specs=[
            pl.BlockSpec((BR, 128), lambda i: (i, 0)),
            pl.BlockSpec((200, 128), lambda i: (0, 0)),
            pl.BlockSpec((200, 128), lambda i: (0, 0)),
            pl.BlockSpec((200, 128), lambda i: (0, 0)),
            pl.BlockSpec((3, B), lambda i: (0, 0)),
        ],
        out_specs=pl.BlockSpec(
            (1, 1), lambda i: (0, 0), memory_space=pltpu.SMEM),
        out_shape=jax.ShapeDtypeStruct((1, 1), jnp.float32),
        compiler_params=pltpu.CompilerParams(
            dimension_semantics=("arbitrary",)),
    )(pred16, c1, c2, c3, obj)
    return out.reshape(())


# speedup vs baseline: 1.7090x; 1.0183x over previous
import dataclasses
import functools

import jax
import jax.numpy as jnp
import numpy as np
from jax import lax
from jax.experimental import pallas as pl
from jax.experimental.pallas import tpu as pltpu
from jax.experimental.pallas import tpu_sc as plsc

GRID = 26
S = GRID * GRID
A = 5
B = 1024
SA = S * A
LANE = SA * A
NROW128 = B * LANE // 128
BR = 10400
NSTEPS = NROW128 // BR

_T = (128 * np.arange(25)[:, None] + np.arange(128)[None, :]) % 25
_CH = _T % 5
_ANC = _T // 5
_C1_NP = np.where(_CH < 2, 1.25 / 2704.0, 0.0).astype(np.float32)
_C3_NP = np.where(_CH == 4, 0.5, 0.0).astype(np.float32)


def _dense_body(pred_ref, cw_ref, ch_ref, obj_ref, out_ref):
    i = pl.program_id(0)
    v0 = pred_ref[0, 0]
    v1 = pred_ref[0, 1]
    v2 = pred_ref[0, 2]
    v3 = pred_ref[0, 3]
    v4 = pred_ref[0, 4]
    r0 = 1.0 / (jnp.exp(v0) + 1.0)
    r1 = 1.0 / (jnp.exp(v1) + 1.0)
    u2 = jnp.exp(v2) - 1.0
    u3 = jnp.exp(v3) - 1.0
    w4 = 1.0 - 1.0 / (jnp.exp(v4) + 1.0)
    s01 = jnp.sum((0.5 - r0) * (0.5 - r0) + (0.5 - r1) * (0.5 - r1))
    s2 = jnp.sum(u2 * u2)
    s3 = jnp.sum(u3 * u3)
    s4 = jnp.sum(w4 * w4)
    part = 1.25 * (s01 * (1.0 / float(S)) + cw_ref[i] * s2 + ch_ref[i] * s3) \
        + 0.5 * s4
    acc = jnp.where(i == 0, 0.0, out_ref[0, 0]) + part

    @pl.when(i < A - 1)
    def _():
        out_ref[0, 0] = acc

    @pl.when(i == A - 1)
    def _():
        ot = obj_ref[...]
        co_sum = jnp.sum(ot[0, :])
        cf_sum = jnp.sum(ot[1, :])
        corr_sum = jnp.sum(ot[2, :])
        out_ref[0, 0] = (5.0 / (4.0 * B)) * co_sum + cf_sum / B \
            + (acc - corr_sum) / (B * (SA - 1))


def _sc_objterms(pred16, lab_t, awp, ahp):
    mesh = plsc.VectorSubcoreMesh(core_axis_name="c", subcore_axis_name="s")
    cp = pltpu.CompilerParams()
    if "needs_layout_passes" in pltpu.CompilerParams.__dataclass_fields__:
        cp = dataclasses.replace(cp, needs_layout_passes=False)

    @functools.partial(
        pl.kernel,
        mesh=mesh,
        compiler_params=cp,
        out_type=jax.ShapeDtypeStruct((3, B), jnp.float32),
        scratch_types=[
            pltpu.VMEM((4, 32), jnp.float32),
            pltpu.VMEM((16,), jnp.float32),
            pltpu.VMEM((16,), jnp.float32),
            pltpu.VMEM((16,), jnp.int32),
            pltpu.VMEM((16,), jnp.int32),
            pltpu.VMEM((16, 128), jnp.float32),
            pltpu.VMEM((16, 128), jnp.float32),
            pltpu.VMEM((3, 32), jnp.float32),
            pltpu.SemaphoreType.DMA,
        ],
    )
    def body(pred_hbm, lab_hbm, aw_hbm, ah_hbm, out_hbm,
             lab_v, aw_v, ah_v, idx_a, idx_b, buf_a, buf_b, out_v, sem):
        wid = lax.axis_index("s") * 2 + lax.axis_index("c")
        base = wid * 32
        for c in range(4):
            pltpu.sync_copy(lab_hbm.at[c, pl.ds(base, 32)], lab_v.at[c])
        pltpu.sync_copy(aw_hbm, aw_v)
        pltpu.sync_copy(ah_hbm, ah_v)
        iota = lax.iota(jnp.int32, 16)

        for g in range(2):
            sl = pl.ds(g * 16, 16)
            lx = lab_v[0, sl]
            ly = lab_v[1, sl]
            lw = lab_v[2, sl]
            lh = lab_v[3, sl]
            ixf = (lx * float(GRID)).astype(jnp.int32)
            iyf = (ly * float(GRID)).astype(jnp.int32)
            s_obj = ixf * GRID + iyf
            awvec = aw_v[...]
            ahvec = ah_v[...]
            best_a = jnp.zeros((16,), jnp.int32)
            best_d = None
            for a in range(A):
                awa = jnp.sum(jnp.where(iota == a, awvec, 0.0))
                aha = jnp.sum(jnp.where(iota == a, ahvec, 0.0))
                d = (lw - awa) * (lw - awa) + (lh - aha) * (lh - aha)
                if best_d is None:
                    best_d = d
                else:
                    m = d > best_d
                    best_a = jnp.where(m, a, best_a)
                    best_d = jnp.where(m, d, best_d)
            slot = s_obj * A + best_a
            b_vec = base + g * 16 + iota
            eb = b_vec * LANE + slot * A
            r0 = eb >> 7
            r1 = jnp.minimum(r0 + 1, NROW128 - 1)
            off = eb - (r0 << 7)
            idx_a[...] = r0
            idx_b[...] = r1
            pltpu.async_copy(pred_hbm.at[idx_a], buf_a, sem).wait()
            pltpu.async_copy(pred_hbm.at[idx_b], buf_b, sem).wait()
            pv = []
            for c in range(5):
                oc = off + c
                va = plsc.load_gather(buf_a, [iota, jnp.minimum(oc, 127)])
                vb = plsc.load_gather(buf_b, [iota, jnp.maximum(oc - 128, 0)])
                pv.append(jnp.where(oc < 128, va, vb))
            p0, p1, p2, p3, p4 = pv
            e0 = jnp.exp(p0)
            e1 = jnp.exp(p1)
            e2 = jnp.exp(p2)
            e3 = jnp.exp(p3)
            e4 = jnp.exp(p4)
            r0f = 1.0 / (e0 + 1.0)
            r1f = 1.0 / (e1 + 1.0)
            sig4 = 1.0 - 1.0 / (e4 + 1.0)
            aw_o = plsc.load_gather(aw_v, [best_a])
            ah_o = plsc.load_gather(ah_v, [best_a])
            px = (ixf.astype(jnp.float32) + (1.0 - r0f)) / float(GRID)
            py = (iyf.astype(jnp.float32) + (1.0 - r1f)) / float(GRID)
            pw = aw_o * e2
            ph = ah_o * e3
            co = ((px - lx) * (px - lx) + (py - ly) * (py - ly)
                  + (pw - lw) * (pw - lw) + (ph - lh) * (ph - lh))
            lx0 = jnp.maximum(lx - lw * 0.5, 0.0)
            ly0 = jnp.maximum(ly - lh * 0.5, 0.0)
            lx1 = jnp.minimum(lx + lw * 0.5, 1.0)
            ly1 = jnp.minimum(ly + lh * 0.5, 1.0)
            px0 = jnp.maximum(px - pw * 0.5, 0.0)
            py0 = jnp.maximum(py - ph * 0.5, 0.0)
            px1 = jnp.minimum(px + pw * 0.5, 1.0)
            py1 = jnp.minimum(py + ph * 0.5, 1.0)
            inter = (jnp.maximum(jnp.minimum(lx1, px1) - jnp.maximum(lx0, px0), 0.0)
                     * jnp.maximum(jnp.minimum(ly1, py1) - jnp.maximum(ly0, py0), 0.0))
            iou = inter / (lw * lh + pw * ph - inter)
            cf = (sig4 - iou) * (sig4 - iou)
            u0 = e0 - 1.0
            u1 = e1 - 1.0
            t0 = u0 * r0f
            t1 = u1 * r1f
            uw = aw_o * (e2 - 1.0)
            uh = ah_o * (e3 - 1.0)
            corr = (1.25 * ((t0 * t0 + t1 * t1) * (1.0 / 2704.0)
                            + uw * uw + uh * uh)
                    + 0.5 * (sig4 * sig4))
            out_v[0, sl] = co
            out_v[1, sl] = cf
            out_v[2, sl] = corr
        for k in range(3):
            pltpu.sync_copy(out_v.at[k], out_hbm.at[k, pl.ds(base, 32)])

    return body(pred16, lab_t, awp, ahp)


def kernel(pred_, label, anchors):
    pred16 = pred_.reshape(NROW128, 128)
    pred_t = jnp.transpose(pred_, (2, 3, 1, 0))
    lab_t = label.T
    aw = anchors[:, 0]
    ah = anchors[:, 1]
    awp = jnp.concatenate([aw, jnp.zeros((11,), jnp.float32)])
    ahp = jnp.concatenate([ah, jnp.zeros((11,), jnp.float32)])

    obj = _sc_objterms(pred16, lab_t, awp, ahp)

    out = pl.pallas_call(
        _dense_body,
        grid=(A,),
        in_specs=[
            pl.BlockSpec((1, A, S, B), lambda i: (i, 0, 0, 0)),
            pl.BlockSpec(memory_space=pltpu.SMEM),
            pl.BlockSpec(memory_space=pltpu.SMEM),
            pl.BlockSpec((3, B), lambda i: (0, 0)),
        ],
        out_specs=pl.BlockSpec(
            (1, 1), lambda i: (0, 0), memory_space=pltpu.SMEM),
        out_shape=jax.ShapeDtypeStruct((1, 1), jnp.float32),
        compiler_params=pltpu.CompilerParams(
            dimension_semantics=("arbitrary",)),
    )(pred_t, aw * aw, ah * ah, obj)
    return out.reshape(())


# speedup vs baseline: 46.0631x; 26.9537x over previous
import dataclasses
import functools

import jax
import jax.numpy as jnp
import numpy as np
from jax import lax
from jax.experimental import pallas as pl
from jax.experimental.pallas import tpu as pltpu
from jax.experimental.pallas import tpu_sc as plsc

GRID = 26
S = GRID * GRID
A = 5
B = 1024
SA = S * A


def _dense_body(pred_ref, sa_ref, lab_ref, aw_ref, ah_ref, cw_ref, ch_ref,
                out_ref, acc_ref):
    i = pl.program_id(0)
    s_obj = sa_ref[0, :]
    a_obj = sa_ref[1, :]
    s_obj_i = s_obj.astype(jnp.int32)
    a_obj_i = a_obj.astype(jnp.int32)
    srow = lax.broadcasted_iota(jnp.int32, (S, B), 0)
    m = jnp.where(
        (srow == s_obj_i[None, :]) & (a_obj_i[None, :] == i), 1.0, 0.0)
    v0 = pred_ref[0, 0]
    v1 = pred_ref[0, 1]
    v2 = pred_ref[0, 2]
    v3 = pred_ref[0, 3]
    v4 = pred_ref[0, 4]
    r0 = 1.0 / (jnp.exp(v0) + 1.0)
    r1 = 1.0 / (jnp.exp(v1) + 1.0)
    u2 = jnp.exp(v2) - 1.0
    u3 = jnp.exp(v3) - 1.0
    w4 = 1.0 - 1.0 / (jnp.exp(v4) + 1.0)
    s01 = jnp.sum((0.5 - r0) * (0.5 - r0) + (0.5 - r1) * (0.5 - r1))
    s2 = jnp.sum(u2 * u2)
    s3 = jnp.sum(u3 * u3)
    s4 = jnp.sum(w4 * w4)
    part = 1.25 * (s01 * (1.0 / float(S)) + cw_ref[i] * s2 + ch_ref[i] * s3) \
        + 0.5 * s4
    acc = jnp.where(i == 0, 0.0, out_ref[0, 0]) + part

    for c, v in enumerate((v0, v1, v2, v3, v4)):
        g = jnp.sum(v * m, axis=0)
        acc_ref[c, :] = jnp.where(i == 0, g, acc_ref[c, :] + g)

    @pl.when(i < A - 1)
    def _():
        out_ref[0, 0] = acc

    @pl.when(i == A - 1)
    def _():
        p0 = acc_ref[0, :]
        p1 = acc_ref[1, :]
        p2 = acc_ref[2, :]
        p3 = acc_ref[3, :]
        p4 = acc_ref[4, :]
        lx = lab_ref[0, :]
        ly = lab_ref[1, :]
        lw = lab_ref[2, :]
        lh = lab_ref[3, :]
        ixf = jnp.floor(s_obj * (1.0 / float(GRID)))
        iyf = s_obj - float(GRID) * ixf
        e0 = jnp.exp(p0)
        e1 = jnp.exp(p1)
        e2 = jnp.exp(p2)
        e3 = jnp.exp(p3)
        e4 = jnp.exp(p4)
        r0f = 1.0 / (e0 + 1.0)
        r1f = 1.0 / (e1 + 1.0)
        sig4 = 1.0 - 1.0 / (e4 + 1.0)
        aw_o = jnp.zeros_like(p0)
        ah_o = jnp.zeros_like(p0)
        for k in range(A):
            sel = a_obj == float(k)
            aw_o = jnp.where(sel, aw_ref[k], aw_o)
            ah_o = jnp.where(sel, ah_ref[k], ah_o)
        px = (ixf + (1.0 - r0f)) * (1.0 / float(GRID))
        py = (iyf + (1.0 - r1f)) * (1.0 / float(GRID))
        pw = aw_o * e2
        ph = ah_o * e3
        co = ((px - lx) * (px - lx) + (py - ly) * (py - ly)
              + (pw - lw) * (pw - lw) + (ph - lh) * (ph - lh))
        lx0 = jnp.maximum(lx - lw * 0.5, 0.0)
        ly0 = jnp.maximum(ly - lh * 0.5, 0.0)
        lx1 = jnp.minimum(lx + lw * 0.5, 1.0)
        ly1 = jnp.minimum(ly + lh * 0.5, 1.0)
        px0 = jnp.maximum(px - pw * 0.5, 0.0)
        py0 = jnp.maximum(py - ph * 0.5, 0.0)
        px1 = jnp.minimum(px + pw * 0.5, 1.0)
        py1 = jnp.minimum(py + ph * 0.5, 1.0)
        inter = (jnp.maximum(jnp.minimum(lx1, px1) - jnp.maximum(lx0, px0), 0.0)
                 * jnp.maximum(jnp.minimum(ly1, py1) - jnp.maximum(ly0, py0),
                               0.0))
        iou = inter / (lw * lh + pw * ph - inter)
        cf = (sig4 - iou) * (sig4 - iou)
        u0 = e0 - 1.0
        u1 = e1 - 1.0
        t0 = u0 * r0f
        t1 = u1 * r1f
        uw = aw_o * (e2 - 1.0)
        uh = ah_o * (e3 - 1.0)
        corr = (1.25 * ((t0 * t0 + t1 * t1) * (1.0 / 2704.0)
                        + uw * uw + uh * uh)
                + 0.5 * (sig4 * sig4))
        out_ref[0, 0] = (5.0 / (4.0 * B)) * jnp.sum(co) + jnp.sum(cf) / B \
            + (acc - jnp.sum(corr)) / (B * (SA - 1))


def _sc_routing(lab_t, awp, ahp):
    mesh = plsc.VectorSubcoreMesh(core_axis_name="c", subcore_axis_name="s")
    cp = pltpu.CompilerParams()
    if "needs_layout_passes" in pltpu.CompilerParams.__dataclass_fields__:
        cp = dataclasses.replace(cp, needs_layout_passes=False)

    @functools.partial(
        pl.kernel,
        mesh=mesh,
        compiler_params=cp,
        out_type=jax.ShapeDtypeStruct((2, B), jnp.float32),
        scratch_types=[
            pltpu.VMEM((4, 32), jnp.float32),
            pltpu.VMEM((16,), jnp.float32),
            pltpu.VMEM((16,), jnp.float32),
            pltpu.VMEM((2, 32), jnp.float32),
        ],
    )
    def body(lab_hbm, aw_hbm, ah_hbm, out_hbm, lab_v, aw_v, ah_v, out_v):
        wid = lax.axis_index("s") * 2 + lax.axis_index("c")
        base = wid * 32
        for c in range(4):
            pltpu.sync_copy(lab_hbm.at[c, pl.ds(base, 32)], lab_v.at[c])
        pltpu.sync_copy(aw_hbm, aw_v)
        pltpu.sync_copy(ah_hbm, ah_v)
        iota = lax.iota(jnp.int32, 16)

        for g in range(2):
            sl = pl.ds(g * 16, 16)
            lx = lab_v[0, sl]
            ly = lab_v[1, sl]
            lw = lab_v[2, sl]
            lh = lab_v[3, sl]
            ixf = (lx * float(GRID)).astype(jnp.int32)
            iyf = (ly * float(GRID)).astype(jnp.int32)
            s_obj = ixf * GRID + iyf
            awvec = aw_v[...]
            ahvec = ah_v[...]
            best_a = jnp.zeros((16,), jnp.int32)
            best_d = None
            for a in range(A):
                awa = jnp.sum(jnp.where(iota == a, awvec, 0.0))
                aha = jnp.sum(jnp.where(iota == a, ahvec, 0.0))
                d = (lw - awa) * (lw - awa) + (lh - aha) * (lh - aha)
                if best_d is None:
                    best_d = d
                else:
                    mgt = d > best_d
                    best_a = jnp.where(mgt, a, best_a)
                    best_d = jnp.where(mgt, d, best_d)
            out_v[0, sl] = s_obj.astype(jnp.float32)
            out_v[1, sl] = best_a.astype(jnp.float32)
        for k in range(2):
            pltpu.sync_copy(out_v.at[k], out_hbm.at[k, pl.ds(base, 32)])

    return body(lab_t, awp, ahp)


def kernel(pred_, label, anchors):
    pred_t = jnp.transpose(pred_, (2, 3, 1, 0))
    lab_t = label.T
    aw = anchors[:, 0]
    ah = anchors[:, 1]
    awp = jnp.concatenate([aw, jnp.zeros((11,), jnp.float32)])
    ahp = jnp.concatenate([ah, jnp.zeros((11,), jnp.float32)])

    sa = _sc_routing(lab_t, awp, ahp)

    out = pl.pallas_call(
        _dense_body,
        grid=(A,),
        in_specs=[
            pl.BlockSpec((1, A, S, B), lambda i: (i, 0, 0, 0)),
            pl.BlockSpec((2, B), lambda i: (0, 0)),
            pl.BlockSpec((4, B), lambda i: (0, 0)),
            pl.BlockSpec(memory_space=pltpu.SMEM),
            pl.BlockSpec(memory_space=pltpu.SMEM),
            pl.BlockSpec(memory_space=pltpu.SMEM),
            pl.BlockSpec(memory_space=pltpu.SMEM),
        ],
        out_specs=pl.BlockSpec(
            (1, 1), lambda i: (0, 0), memory_space=pltpu.SMEM),
        out_shape=jax.ShapeDtypeStruct((1, 1), jnp.float32),
        scratch_shapes=[pltpu.VMEM((8, B), jnp.float32)],
        compiler_params=pltpu.CompilerParams(
            dimension_semantics=("arbitrary",)),
    )(pred_t, sa, lab_t, aw, ah, aw * aw, ah * ah)
    return out.reshape(())
